# Initial kernel scaffold; baseline (speedup 1.0000x reference)
import jax
import jax.numpy as jnp
from jax.experimental import pallas as pl


def kernel(x, edge_index, W1, b1, W2, b2, W3, b3, W4, b4, fc1_W, fc1_b, fc2_W, fc2_b):
    raise NotImplementedError("write your pallas kernel here")


# speedup vs baseline: 14.7959x; 14.7959x over previous
import functools

import jax
import jax.numpy as jnp
from jax import lax
from jax.experimental import pallas as pl
from jax.experimental.pallas import tpu as pltpu
from jax.experimental.pallas import tpu_sc as plsc

N = 10000
E = 320000
D_IN = 128
D_H = 64
CHUNK = 128
NCHUNKS = E // CHUNK
NC = 2
NS = 16
NW = NC * NS
NPAD = 10240
ROWS_PER_TILE = NPAD // NS

_mesh = plsc.VectorSubcoreMesh(
    core_axis_name="c", subcore_axis_name="s", num_cores=NC, num_subcores=NS
)


def _leaky(v):
    return jnp.where(v >= 0, v, 0.01 * v)


DEG_W = 16


def _sc_deg_body(dst2d, zeros_hbm, ones_hbm, out, didx, ones_v, zbuf, acc):
    c = lax.axis_index("c")
    s = lax.axis_index("s")
    w = s * NC + c

    pltpu.sync_copy(ones_hbm, ones_v)
    pltpu.sync_copy(zeros_hbm, zbuf)
    pltpu.sync_copy(zbuf, acc.at[pl.ds(s * ROWS_PER_TILE, ROWS_PER_TILE)])
    plsc.subcore_barrier()

    def cbody(j, carry):
        g = j * NW + w

        @pl.when(g < NCHUNKS)
        def _():
            pltpu.sync_copy(dst2d.at[g], didx)
            pltpu.sync_copy(ones_v, acc.at[didx], add=True)

        return carry

    lax.fori_loop(0, (NCHUNKS + NW - 1) // NW, cbody, 0)
    plsc.subcore_barrier()
    pltpu.sync_copy(
        acc.at[pl.ds(s * ROWS_PER_TILE, ROWS_PER_TILE)],
        out.at[c, pl.ds(s * ROWS_PER_TILE, ROWS_PER_TILE)],
    )


_sc_deg = pl.kernel(
    _sc_deg_body,
    out_type=jax.ShapeDtypeStruct((NC, NPAD, DEG_W), jnp.float32),
    mesh=_mesh,
    compiler_params=pltpu.CompilerParams(use_tc_tiling_on_sc=False),
    scratch_types=[
        pltpu.VMEM((CHUNK,), jnp.int32),
        pltpu.VMEM((CHUNK, DEG_W), jnp.float32),
        pltpu.VMEM((ROWS_PER_TILE, DEG_W), jnp.float32),
        pltpu.VMEM_SHARED((NPAD, DEG_W), jnp.float32),
    ],
)


def _sc_scat_body(hp, src2d, dst2d, zeros_hbm, out, sidx, didx, rows, zbuf,
                  acc, gsem):
    c = lax.axis_index("c")
    s = lax.axis_index("s")
    w = s * NC + c

    pltpu.sync_copy(zeros_hbm, zbuf)
    pltpu.sync_copy(zbuf, acc.at[pl.ds(s * ROWS_PER_TILE, ROWS_PER_TILE)])
    plsc.subcore_barrier()

    def cbody(j, carry):
        g = j * NW + w

        @pl.when(g < NCHUNKS)
        def _():
            pltpu.sync_copy(src2d.at[g], sidx)
            pltpu.sync_copy(dst2d.at[g], didx)
            pltpu.async_copy(hp.at[sidx], rows, gsem).wait()
            pltpu.sync_copy(rows, acc.at[didx], add=True)

        return carry

    lax.fori_loop(0, (NCHUNKS + NW - 1) // NW, cbody, 0)
    plsc.subcore_barrier()
    pltpu.sync_copy(
        acc.at[pl.ds(s * ROWS_PER_TILE, ROWS_PER_TILE)],
        out.at[c, pl.ds(s * ROWS_PER_TILE, ROWS_PER_TILE)],
    )


_sc_scat = pl.kernel(
    _sc_scat_body,
    out_type=jax.ShapeDtypeStruct((NC, NPAD, D_H), jnp.float32),
    mesh=_mesh,
    compiler_params=pltpu.CompilerParams(use_tc_tiling_on_sc=False),
    scratch_types=[
        pltpu.VMEM((CHUNK,), jnp.int32),
        pltpu.VMEM((CHUNK,), jnp.int32),
        pltpu.VMEM((CHUNK, D_H), jnp.float32),
        pltpu.VMEM((ROWS_PER_TILE, D_H), jnp.float32),
        pltpu.VMEM_SHARED((NPAD, D_H), jnp.float32),
        pltpu.SemaphoreType.DMA,
    ],
)


def _tc_first_body(degp_ref, x_ref, w1_ref, hp_ref, dinv_ref):
    deg = degp_ref[0, 0:N, 0] + degp_ref[1, 0:N, 0] + 1.0
    dinv = lax.rsqrt(deg).reshape(N, 1)
    h = jnp.dot(x_ref[...], w1_ref[...], preferred_element_type=jnp.float32)
    hp_ref[...] = h * dinv
    dinv_ref[...] = dinv


_tc_first = pl.pallas_call(
    _tc_first_body,
    out_shape=(
        jax.ShapeDtypeStruct((N, D_H), jnp.float32),
        jax.ShapeDtypeStruct((N, 1), jnp.float32),
    ),
)


def _tc_mid_body(parts_ref, hp_ref, dinv_ref, b_ref, w_ref, hpn_ref):
    dinv = dinv_ref[...]
    agg = parts_ref[0, 0:N] + parts_ref[1, 0:N] + hp_ref[...]
    xl = _leaky(agg * dinv + b_ref[...])
    h = jnp.dot(xl, w_ref[...], preferred_element_type=jnp.float32)
    hpn_ref[...] = h * dinv


_tc_mid = pl.pallas_call(
    _tc_mid_body,
    out_shape=jax.ShapeDtypeStruct((N, D_H), jnp.float32),
)


def _tc_final_body(parts_ref, hp_ref, dinv_ref, b_ref, fc1w_ref, fc1b_ref,
                   fc2w_ref, fc2b_ref, out_ref):
    dinv = dinv_ref[...]
    agg = parts_ref[0, 0:N] + parts_ref[1, 0:N] + hp_ref[...]
    xl = _leaky(agg * dinv + b_ref[...])
    g = jnp.mean(xl, axis=0, keepdims=True)
    z = _leaky(
        jnp.dot(g, fc1w_ref[...], preferred_element_type=jnp.float32)
        + fc1b_ref[...]
    )
    z = (
        jnp.dot(z, fc2w_ref[...], preferred_element_type=jnp.float32)
        + fc2b_ref[...]
    )
    out_ref[...] = 1.0 / (1.0 + jnp.exp(-z))


_tc_final = pl.pallas_call(
    _tc_final_body,
    out_shape=jax.ShapeDtypeStruct((1, 1), jnp.float32),
)


def kernel(x, edge_index, W1, b1, W2, b2, W3, b3, W4, b4,
           fc1_W, fc1_b, fc2_W, fc2_b):
    src2d = edge_index[0].reshape(NCHUNKS, CHUNK)
    dst2d = edge_index[1].reshape(NCHUNKS, CHUNK)
    zeros_deg = jnp.zeros((ROWS_PER_TILE, DEG_W), jnp.float32)
    ones_deg = jnp.ones((CHUNK, DEG_W), jnp.float32)
    zeros_h = jnp.zeros((ROWS_PER_TILE, D_H), jnp.float32)

    degp = _sc_deg(dst2d, zeros_deg, ones_deg)
    hp, dinv = _tc_first(degp, x, W1)
    for b, w in ((b1, W2), (b2, W3), (b3, W4)):
        parts = _sc_scat(hp, src2d, dst2d, zeros_h)
        hp = _tc_mid(parts, hp, dinv, b.reshape(1, D_H), w)
    parts = _sc_scat(hp, src2d, dst2d, zeros_h)
    return _tc_final(
        parts, hp, dinv, b4.reshape(1, D_H),
        fc1_W, fc1_b.reshape(1, D_H), fc2_W, fc2_b.reshape(1, 1),
    )


# speedup vs baseline: 31.2118x; 2.1095x over previous
import jax
import jax.numpy as jnp
from jax import lax
from jax.experimental import pallas as pl
from jax.experimental.pallas import tpu as pltpu
from jax.experimental.pallas import tpu_sc as plsc

N = 10000
E = 320000
D_IN = 128
D_H = 64
CHUNK = 128
NCHUNKS = E // CHUNK
NC = 2
NS = 16
NW = NC * NS
NPAD = 10240
ROWS_PER_TILE = NPAD // NS
CPT = NCHUNKS // NW
CPT_REM = NCHUNKS % NW
MAXCPT = CPT + 1
NBUF = 8
TOUT = (MAXCPT + NBUF - 1) // NBUF

_mesh = plsc.VectorSubcoreMesh(
    core_axis_name="c", subcore_axis_name="s", num_cores=NC, num_subcores=NS
)
_sc_params = pltpu.CompilerParams(use_tc_tiling_on_sc=False)


def _leaky(v):
    return jnp.where(v >= 0, v, 0.01 * v)


def _tile_chunk_range(w):
    count = jnp.where(w < CPT_REM, CPT + 1, CPT)
    start = CPT * w + jnp.minimum(w, CPT_REM)
    return start, count


def _preload_idx(src2d, start, count, idx_all):
    pltpu.sync_copy(src2d.at[pl.ds(start, CPT)], idx_all.at[pl.ds(0, CPT)])

    @pl.when(count > CPT)
    def _():
        pltpu.sync_copy(src2d.at[start + CPT], idx_all.at[CPT])


DEG_W = 16


def _sc_deg_body(dst2d, zeros_hbm, ones_hbm, out, didx_all, ones_v, zbuf,
                 acc, ssem):
    c = lax.axis_index("c")
    s = lax.axis_index("s")
    w = s * NC + c
    start, count = _tile_chunk_range(w)

    pltpu.sync_copy(ones_hbm, ones_v)
    pltpu.sync_copy(zeros_hbm, zbuf)
    pltpu.sync_copy(zbuf, acc.at[pl.ds(s * ROWS_PER_TILE, ROWS_PER_TILE)])
    _preload_idx(dst2d, start, count, didx_all)
    plsc.subcore_barrier()

    def obody(k, carry):
        base = k * NBUF
        for b in range(NBUF):
            j = base + b

            @pl.when(j < count)
            def _(b=b, j=j):
                pltpu.async_copy(
                    ones_v, acc.at[didx_all.at[j]], ssem.at[b], add=True
                )

        for b in range(NBUF):
            j = base + b
            desc = pltpu.make_async_copy(
                ones_v, acc.at[didx_all.at[j]], ssem.at[b]
            )
            pl.when(j < count)(desc.wait)
        return carry

    lax.fori_loop(0, TOUT, obody, 0)
    plsc.subcore_barrier()
    pltpu.sync_copy(
        acc.at[pl.ds(s * ROWS_PER_TILE, ROWS_PER_TILE)],
        out.at[c, pl.ds(s * ROWS_PER_TILE, ROWS_PER_TILE)],
    )


_sc_deg = pl.kernel(
    _sc_deg_body,
    out_type=jax.ShapeDtypeStruct((NC, NPAD, DEG_W), jnp.float32),
    mesh=_mesh,
    compiler_params=_sc_params,
    scratch_types=[
        pltpu.VMEM((MAXCPT, CHUNK), jnp.int32),
        pltpu.VMEM((CHUNK, DEG_W), jnp.float32),
        pltpu.VMEM((ROWS_PER_TILE, DEG_W), jnp.float32),
        pltpu.VMEM_SHARED((NPAD, DEG_W), jnp.float32),
        pltpu.SemaphoreType.DMA((NBUF,)),
    ],
)


def _sc_scat_body(hp, src2d, dst2d, zeros_hbm, out, sidx_all, didx_all, rows,
                  acc, gsem):
    c = lax.axis_index("c")
    s = lax.axis_index("s")
    w = s * NC + c
    start, count = _tile_chunk_range(w)

    pltpu.sync_copy(zeros_hbm, rows.at[0])
    for i in range(ROWS_PER_TILE // CHUNK):
        pltpu.sync_copy(
            rows.at[0], acc.at[pl.ds(s * ROWS_PER_TILE + i * CHUNK, CHUNK)]
        )
    _preload_idx(src2d, start, count, sidx_all)
    _preload_idx(dst2d, start, count, didx_all)
    plsc.subcore_barrier()

    def obody(k, carry):
        base = k * NBUF
        for b in range(NBUF):
            j = base + b
            desc = pltpu.make_async_copy(
                hp.at[sidx_all.at[j]], rows.at[b], gsem.at[b]
            )
            pl.when(j < count)(desc.start)
        for b in range(NBUF):
            j = base + b

            @pl.when(j < count)
            def _(b=b, j=j):
                pltpu.make_async_copy(
                    hp.at[sidx_all.at[j]], rows.at[b], gsem.at[b]
                ).wait()
                pltpu.sync_copy(rows.at[b], acc.at[didx_all.at[j]], add=True)

        return carry

    lax.fori_loop(0, TOUT, obody, 0)
    plsc.subcore_barrier()
    pltpu.sync_copy(
        acc.at[pl.ds(s * ROWS_PER_TILE, ROWS_PER_TILE)],
        out.at[c, pl.ds(s * ROWS_PER_TILE, ROWS_PER_TILE)],
    )


_sc_scat = pl.kernel(
    _sc_scat_body,
    out_type=jax.ShapeDtypeStruct((NC, NPAD, D_H), jnp.float32),
    mesh=_mesh,
    compiler_params=_sc_params,
    scratch_types=[
        pltpu.VMEM((MAXCPT, CHUNK), jnp.int32),
        pltpu.VMEM((MAXCPT, CHUNK), jnp.int32),
        pltpu.VMEM((NBUF, CHUNK, D_H), jnp.float32),
        pltpu.VMEM_SHARED((NPAD, D_H), jnp.float32),
        pltpu.SemaphoreType.DMA((NBUF,)),
    ],
)


def _tc_first_body(degp_ref, x_ref, w1_ref, hp_ref, dinv_ref):
    deg = degp_ref[0, 0:N, 0] + degp_ref[1, 0:N, 0] + 1.0
    dinv = lax.rsqrt(deg).reshape(N, 1)
    h = jnp.dot(x_ref[...], w1_ref[...], preferred_element_type=jnp.float32)
    hp_ref[...] = h * dinv
    dinv_ref[...] = dinv


_tc_first = pl.pallas_call(
    _tc_first_body,
    out_shape=(
        jax.ShapeDtypeStruct((N, D_H), jnp.float32),
        jax.ShapeDtypeStruct((N, 1), jnp.float32),
    ),
)


def _tc_mid_body(parts_ref, hp_ref, dinv_ref, b_ref, w_ref, hpn_ref):
    dinv = dinv_ref[...]
    agg = parts_ref[0, 0:N] + parts_ref[1, 0:N] + hp_ref[...]
    xl = _leaky(agg * dinv + b_ref[...])
    h = jnp.dot(xl, w_ref[...], preferred_element_type=jnp.float32)
    hpn_ref[...] = h * dinv


_tc_mid = pl.pallas_call(
    _tc_mid_body,
    out_shape=jax.ShapeDtypeStruct((N, D_H), jnp.float32),
)


def _tc_final_body(parts_ref, hp_ref, dinv_ref, b_ref, fc1w_ref, fc1b_ref,
                   fc2w_ref, fc2b_ref, out_ref):
    dinv = dinv_ref[...]
    agg = parts_ref[0, 0:N] + parts_ref[1, 0:N] + hp_ref[...]
    xl = _leaky(agg * dinv + b_ref[...])
    g = jnp.mean(xl, axis=0, keepdims=True)
    z = _leaky(
        jnp.dot(g, fc1w_ref[...], preferred_element_type=jnp.float32)
        + fc1b_ref[...]
    )
    z = (
        jnp.dot(z, fc2w_ref[...], preferred_element_type=jnp.float32)
        + fc2b_ref[...]
    )
    out_ref[...] = 1.0 / (1.0 + jnp.exp(-z))


_tc_final = pl.pallas_call(
    _tc_final_body,
    out_shape=jax.ShapeDtypeStruct((1, 1), jnp.float32),
)


def kernel(x, edge_index, W1, b1, W2, b2, W3, b3, W4, b4,
           fc1_W, fc1_b, fc2_W, fc2_b):
    src2d = edge_index[0].reshape(NCHUNKS, CHUNK)
    dst2d = edge_index[1].reshape(NCHUNKS, CHUNK)
    zeros_deg = jnp.zeros((ROWS_PER_TILE, DEG_W), jnp.float32)
    ones_deg = jnp.ones((CHUNK, DEG_W), jnp.float32)
    zeros_h = jnp.zeros((CHUNK, D_H), jnp.float32)

    degp = _sc_deg(dst2d, zeros_deg, ones_deg)
    hp, dinv = _tc_first(degp, x, W1)
    for b, w in ((b1, W2), (b2, W3), (b3, W4)):
        parts = _sc_scat(hp, src2d, dst2d, zeros_h)
        hp = _tc_mid(parts, hp, dinv, b.reshape(1, D_H), w)
    parts = _sc_scat(hp, src2d, dst2d, zeros_h)
    return _tc_final(
        parts, hp, dinv, b4.reshape(1, D_H),
        fc1_W, fc1_b.reshape(1, D_H), fc2_W, fc2_b.reshape(1, 1),
    )


# speedup vs baseline: 35.9636x; 1.1522x over previous
import jax
import jax.numpy as jnp
from jax import lax
from jax.experimental import pallas as pl
from jax.experimental.pallas import tpu as pltpu
from jax.experimental.pallas import tpu_sc as plsc

N = 10000
E = 320000
D_IN = 128
D_H = 64
CHUNK = 128
NCHUNKS = E // CHUNK
NC = 2
NS = 16
NW = NC * NS
NPAD = 10240
ROWS_PER_TILE = NPAD // NS
CPT = NCHUNKS // NW
CPT_REM = NCHUNKS % NW
MAXCPT = CPT + 1
NBUF = 8
TOUT = (MAXCPT + NBUF - 1) // NBUF

_mesh = plsc.VectorSubcoreMesh(
    core_axis_name="c", subcore_axis_name="s", num_cores=NC, num_subcores=NS
)
_sc_params = pltpu.CompilerParams(use_tc_tiling_on_sc=False)


def _leaky(v):
    return jnp.where(v >= 0, v, 0.01 * v)


def _tile_chunk_range(w):
    count = jnp.where(w < CPT_REM, CPT + 1, CPT)
    start = CPT * w + jnp.minimum(w, CPT_REM)
    return start, count


def _preload_idx(src2d, start, count, idx_all):
    pltpu.sync_copy(src2d.at[pl.ds(start, CPT)], idx_all.at[pl.ds(0, CPT)])

    @pl.when(count > CPT)
    def _():
        pltpu.sync_copy(src2d.at[start + CPT], idx_all.at[CPT])


DEG_W = 16


def _sc_deg_body(dst2d, zeros_hbm, ones_hbm, out, didx_all, ones_v, zbuf,
                 acc, ssem):
    c = lax.axis_index("c")
    s = lax.axis_index("s")
    w = s * NC + c
    start, count = _tile_chunk_range(w)

    pltpu.sync_copy(ones_hbm, ones_v)
    pltpu.sync_copy(zeros_hbm, zbuf)
    pltpu.sync_copy(zbuf, acc.at[pl.ds(s * ROWS_PER_TILE, ROWS_PER_TILE)])
    _preload_idx(dst2d, start, count, didx_all)
    plsc.subcore_barrier()

    def obody(k, carry):
        base = k * NBUF
        for b in range(NBUF):
            j = base + b

            @pl.when(jnp.logical_and(k > 0, j < count))
            def _(b=b, j=j):
                pltpu.make_async_copy(
                    ones_v, acc.at[didx_all.at[j - NBUF]], ssem.at[b]
                ).wait()

            @pl.when(j < count)
            def _(b=b, j=j):
                pltpu.async_copy(
                    ones_v, acc.at[didx_all.at[j]], ssem.at[b], add=True
                )

        return carry

    lax.fori_loop(0, TOUT, obody, 0)
    for b in range(NBUF):
        pltpu.make_async_copy(ones_v, acc.at[didx_all.at[b]], ssem.at[b]).wait()
    plsc.subcore_barrier()
    pltpu.sync_copy(
        acc.at[pl.ds(s * ROWS_PER_TILE, ROWS_PER_TILE)],
        out.at[c, pl.ds(s * ROWS_PER_TILE, ROWS_PER_TILE)],
    )


_sc_deg = pl.kernel(
    _sc_deg_body,
    out_type=jax.ShapeDtypeStruct((NC, NPAD, DEG_W), jnp.float32),
    mesh=_mesh,
    compiler_params=_sc_params,
    scratch_types=[
        pltpu.VMEM((MAXCPT, CHUNK), jnp.int32),
        pltpu.VMEM((CHUNK, DEG_W), jnp.float32),
        pltpu.VMEM((ROWS_PER_TILE, DEG_W), jnp.float32),
        pltpu.VMEM_SHARED((NPAD, DEG_W), jnp.float32),
        pltpu.SemaphoreType.DMA((NBUF,)),
    ],
)


def _sc_scat_body(hp, src2d, dst2d, zeros_hbm, out, sidx_all, didx_all, rows,
                  acc, gsem, ssem):
    c = lax.axis_index("c")
    s = lax.axis_index("s")
    w = s * NC + c
    start, count = _tile_chunk_range(w)

    pltpu.sync_copy(zeros_hbm, rows.at[0])
    for i in range(ROWS_PER_TILE // CHUNK):
        pltpu.sync_copy(
            rows.at[0], acc.at[pl.ds(s * ROWS_PER_TILE + i * CHUNK, CHUNK)]
        )
    _preload_idx(src2d, start, count, sidx_all)
    _preload_idx(dst2d, start, count, didx_all)
    plsc.subcore_barrier()

    def obody(k, carry):
        base = k * NBUF
        for b in range(NBUF):
            j = base + b

            @pl.when(jnp.logical_and(k > 0, j < count))
            def _(b=b, j=j):
                pltpu.make_async_copy(
                    rows.at[b], acc.at[didx_all.at[j - NBUF]], ssem.at[b]
                ).wait()

            @pl.when(j < count)
            def _(b=b, j=j):
                pltpu.async_copy(hp.at[sidx_all.at[j]], rows.at[b], gsem.at[b])

        for b in range(NBUF):
            j = base + b

            @pl.when(j < count)
            def _(b=b, j=j):
                pltpu.make_async_copy(
                    hp.at[sidx_all.at[j]], rows.at[b], gsem.at[b]
                ).wait()
                pltpu.async_copy(
                    rows.at[b], acc.at[didx_all.at[j]], ssem.at[b], add=True
                )

        return carry

    lax.fori_loop(0, TOUT, obody, 0)
    for b in range(NBUF):
        pltpu.make_async_copy(
            rows.at[b], acc.at[didx_all.at[b]], ssem.at[b]
        ).wait()
    plsc.subcore_barrier()
    pltpu.sync_copy(
        acc.at[pl.ds(s * ROWS_PER_TILE, ROWS_PER_TILE)],
        out.at[c, pl.ds(s * ROWS_PER_TILE, ROWS_PER_TILE)],
    )


_sc_scat = pl.kernel(
    _sc_scat_body,
    out_type=jax.ShapeDtypeStruct((NC, NPAD, D_H), jnp.float32),
    mesh=_mesh,
    compiler_params=_sc_params,
    scratch_types=[
        pltpu.VMEM((MAXCPT, CHUNK), jnp.int32),
        pltpu.VMEM((MAXCPT, CHUNK), jnp.int32),
        pltpu.VMEM((NBUF, CHUNK, D_H), jnp.float32),
        pltpu.VMEM_SHARED((NPAD, D_H), jnp.float32),
        pltpu.SemaphoreType.DMA((NBUF,)),
        pltpu.SemaphoreType.DMA((NBUF,)),
    ],
)


def _tc_first_body(degp_ref, x_ref, w1_ref, hp_ref, dinv_ref):
    deg = degp_ref[0, 0:N, 0] + degp_ref[1, 0:N, 0] + 1.0
    dinv = lax.rsqrt(deg).reshape(N, 1)
    h = jnp.dot(x_ref[...], w1_ref[...], preferred_element_type=jnp.float32)
    hp_ref[...] = h * dinv
    dinv_ref[...] = dinv


_tc_first = pl.pallas_call(
    _tc_first_body,
    out_shape=(
        jax.ShapeDtypeStruct((N, D_H), jnp.float32),
        jax.ShapeDtypeStruct((N, 1), jnp.float32),
    ),
)


def _tc_mid_body(parts_ref, hp_ref, dinv_ref, b_ref, w_ref, hpn_ref):
    dinv = dinv_ref[...]
    agg = parts_ref[0, 0:N] + parts_ref[1, 0:N] + hp_ref[...]
    xl = _leaky(agg * dinv + b_ref[...])
    h = jnp.dot(xl, w_ref[...], preferred_element_type=jnp.float32)
    hpn_ref[...] = h * dinv


_tc_mid = pl.pallas_call(
    _tc_mid_body,
    out_shape=jax.ShapeDtypeStruct((N, D_H), jnp.float32),
)


def _tc_final_body(parts_ref, hp_ref, dinv_ref, b_ref, fc1w_ref, fc1b_ref,
                   fc2w_ref, fc2b_ref, out_ref):
    dinv = dinv_ref[...]
    agg = parts_ref[0, 0:N] + parts_ref[1, 0:N] + hp_ref[...]
    xl = _leaky(agg * dinv + b_ref[...])
    g = jnp.mean(xl, axis=0, keepdims=True)
    z = _leaky(
        jnp.dot(g, fc1w_ref[...], preferred_element_type=jnp.float32)
        + fc1b_ref[...]
    )
    z = (
        jnp.dot(z, fc2w_ref[...], preferred_element_type=jnp.float32)
        + fc2b_ref[...]
    )
    out_ref[...] = 1.0 / (1.0 + jnp.exp(-z))


_tc_final = pl.pallas_call(
    _tc_final_body,
    out_shape=jax.ShapeDtypeStruct((1, 1), jnp.float32),
)


def kernel(x, edge_index, W1, b1, W2, b2, W3, b3, W4, b4,
           fc1_W, fc1_b, fc2_W, fc2_b):
    src2d = edge_index[0].reshape(NCHUNKS, CHUNK)
    dst2d = edge_index[1].reshape(NCHUNKS, CHUNK)
    zeros_deg = jnp.zeros((ROWS_PER_TILE, DEG_W), jnp.float32)
    ones_deg = jnp.ones((CHUNK, DEG_W), jnp.float32)
    zeros_h = jnp.zeros((CHUNK, D_H), jnp.float32)

    degp = _sc_deg(dst2d, zeros_deg, ones_deg)
    hp, dinv = _tc_first(degp, x, W1)
    for b, w in ((b1, W2), (b2, W3), (b3, W4)):
        parts = _sc_scat(hp, src2d, dst2d, zeros_h)
        hp = _tc_mid(parts, hp, dinv, b.reshape(1, D_H), w)
    parts = _sc_scat(hp, src2d, dst2d, zeros_h)
    return _tc_final(
        parts, hp, dinv, b4.reshape(1, D_H),
        fc1_W, fc1_b.reshape(1, D_H), fc2_W, fc2_b.reshape(1, 1),
    )


# speedup vs baseline: 46.8259x; 1.3020x over previous
import jax
import jax.numpy as jnp
from jax import lax
from jax.experimental import pallas as pl
from jax.experimental.pallas import tpu as pltpu
from jax.experimental.pallas import tpu_sc as plsc

N = 10000
E = 320000
D_IN = 128
D_H = 64
CHUNK = 128
NCHUNKS = E // CHUNK
NC = 2
NS = 16
NW = NC * NS
NPAD = 10240
ROWS_PER_TILE = NPAD // NS
CPT = NCHUNKS // NW
CPT_REM = NCHUNKS % NW
MAXCPT = CPT + 1
NBUF = 8
TOUT = (MAXCPT + NBUF - 1) // NBUF

_mesh = plsc.VectorSubcoreMesh(
    core_axis_name="c", subcore_axis_name="s", num_cores=NC, num_subcores=NS
)
_sc_params = pltpu.CompilerParams(use_tc_tiling_on_sc=False)


def _leaky(v):
    return jnp.where(v >= 0, v, 0.01 * v)


def _tile_chunk_range(w):
    count = jnp.where(w < CPT_REM, CPT + 1, CPT)
    start = CPT * w + jnp.minimum(w, CPT_REM)
    return start, count


def _preload_idx(src2d, start, count, idx_all):
    pltpu.sync_copy(src2d.at[pl.ds(start, CPT)], idx_all.at[pl.ds(0, CPT)])

    @pl.when(count > CPT)
    def _():
        pltpu.sync_copy(src2d.at[start + CPT], idx_all.at[CPT])


DEG_W = 16


def _sc_deg_body(dst2d, zeros_hbm, ones_hbm, out, didx_all, ones_v, zbuf,
                 acc, ssem):
    c = lax.axis_index("c")
    s = lax.axis_index("s")
    w = s * NC + c
    start, count = _tile_chunk_range(w)

    pltpu.sync_copy(ones_hbm, ones_v)
    pltpu.sync_copy(zeros_hbm, zbuf)
    pltpu.sync_copy(zbuf, acc.at[pl.ds(s * ROWS_PER_TILE, ROWS_PER_TILE)])
    _preload_idx(dst2d, start, count, didx_all)
    plsc.subcore_barrier()

    def obody(k, carry):
        base = k * NBUF
        for b in range(NBUF):
            j = base + b

            @pl.when(jnp.logical_and(k > 0, j < count))
            def _(b=b, j=j):
                pltpu.make_async_copy(
                    ones_v, acc.at[didx_all.at[j - NBUF]], ssem.at[b]
                ).wait()

            @pl.when(j < count)
            def _(b=b, j=j):
                pltpu.async_copy(
                    ones_v, acc.at[didx_all.at[j]], ssem.at[b], add=True
                )

        return carry

    lax.fori_loop(0, TOUT, obody, 0)
    for b in range(NBUF):
        pltpu.make_async_copy(ones_v, acc.at[didx_all.at[b]], ssem.at[b]).wait()
    plsc.subcore_barrier()
    pltpu.sync_copy(
        acc.at[pl.ds(s * ROWS_PER_TILE, ROWS_PER_TILE)],
        out.at[c, pl.ds(s * ROWS_PER_TILE, ROWS_PER_TILE)],
    )


_sc_deg = pl.kernel(
    _sc_deg_body,
    out_type=jax.ShapeDtypeStruct((NC, NPAD, DEG_W), jnp.float32),
    mesh=_mesh,
    compiler_params=_sc_params,
    scratch_types=[
        pltpu.VMEM((MAXCPT, CHUNK), jnp.int32),
        pltpu.VMEM((CHUNK, DEG_W), jnp.float32),
        pltpu.VMEM((ROWS_PER_TILE, DEG_W), jnp.float32),
        pltpu.VMEM_SHARED((NPAD, DEG_W), jnp.float32),
        pltpu.SemaphoreType.DMA((NBUF,)),
    ],
)


def _sc_scat_body(hp, src2d, dst2d, zeros_hbm, out, sidx_all, didx_all, rows,
                  acc, gsem, ssem):
    c = lax.axis_index("c")
    s = lax.axis_index("s")
    w = s * NC + c
    start, count = _tile_chunk_range(w)

    pltpu.sync_copy(zeros_hbm, rows.at[0])
    for i in range(ROWS_PER_TILE // CHUNK):
        pltpu.sync_copy(
            rows.at[0], acc.at[pl.ds(s * ROWS_PER_TILE + i * CHUNK, CHUNK)]
        )
    _preload_idx(src2d, start, count, sidx_all)
    _preload_idx(dst2d, start, count, didx_all)
    plsc.subcore_barrier()

    def obody(k, carry):
        base = k * NBUF
        for b in range(NBUF):
            j = base + b

            @pl.when(jnp.logical_and(k > 0, j < count))
            def _(b=b, j=j):
                pltpu.make_async_copy(
                    rows.at[b], acc.at[didx_all.at[j - NBUF]], ssem.at[b]
                ).wait()

            @pl.when(j < count)
            def _(b=b, j=j):
                pltpu.async_copy(hp.at[sidx_all.at[j]], rows.at[b], gsem.at[b])

        for b in range(NBUF):
            j = base + b

            @pl.when(j < count)
            def _(b=b, j=j):
                pltpu.make_async_copy(
                    hp.at[sidx_all.at[j]], rows.at[b], gsem.at[b]
                ).wait()
                pltpu.async_copy(
                    rows.at[b], acc.at[didx_all.at[j]], ssem.at[b], add=True
                )

        return carry

    lax.fori_loop(0, TOUT, obody, 0)
    for b in range(NBUF):
        pltpu.make_async_copy(
            rows.at[b], acc.at[didx_all.at[b]], ssem.at[b]
        ).wait()
    plsc.subcore_barrier()
    pltpu.sync_copy(
        acc.at[pl.ds(s * ROWS_PER_TILE, ROWS_PER_TILE)],
        out.at[c, pl.ds(s * ROWS_PER_TILE, ROWS_PER_TILE)],
    )


_sc_scat = pl.kernel(
    _sc_scat_body,
    out_type=jax.ShapeDtypeStruct((NC, NPAD, D_H), jnp.bfloat16),
    mesh=_mesh,
    compiler_params=_sc_params,
    scratch_types=[
        pltpu.VMEM((MAXCPT, CHUNK), jnp.int32),
        pltpu.VMEM((MAXCPT, CHUNK), jnp.int32),
        pltpu.VMEM((NBUF, CHUNK, D_H), jnp.bfloat16),
        pltpu.VMEM_SHARED((NPAD, D_H), jnp.bfloat16),
        pltpu.SemaphoreType.DMA((NBUF,)),
        pltpu.SemaphoreType.DMA((NBUF,)),
    ],
)


def _tc_first_body(degp_ref, x_ref, w1_ref, hp_ref, dinv_ref):
    deg = degp_ref[0, 0:N, 0] + degp_ref[1, 0:N, 0] + 1.0
    dinv = lax.rsqrt(deg).reshape(N, 1)
    h = jnp.dot(x_ref[...], w1_ref[...], preferred_element_type=jnp.float32)
    hp_ref[...] = (h * dinv).astype(jnp.bfloat16)
    dinv_ref[...] = dinv


_tc_first = pl.pallas_call(
    _tc_first_body,
    out_shape=(
        jax.ShapeDtypeStruct((N, D_H), jnp.bfloat16),
        jax.ShapeDtypeStruct((N, 1), jnp.float32),
    ),
)


def _tc_mid_body(parts_ref, hp_ref, dinv_ref, b_ref, w_ref, hpn_ref):
    dinv = dinv_ref[...]
    agg = (parts_ref[0, 0:N].astype(jnp.float32)
           + parts_ref[1, 0:N].astype(jnp.float32)
           + hp_ref[...].astype(jnp.float32))
    xl = _leaky(agg * dinv + b_ref[...])
    h = jnp.dot(xl, w_ref[...], preferred_element_type=jnp.float32)
    hpn_ref[...] = (h * dinv).astype(jnp.bfloat16)


_tc_mid = pl.pallas_call(
    _tc_mid_body,
    out_shape=jax.ShapeDtypeStruct((N, D_H), jnp.bfloat16),
)


def _tc_final_body(parts_ref, hp_ref, dinv_ref, b_ref, fc1w_ref, fc1b_ref,
                   fc2w_ref, fc2b_ref, out_ref):
    dinv = dinv_ref[...]
    agg = (parts_ref[0, 0:N].astype(jnp.float32)
           + parts_ref[1, 0:N].astype(jnp.float32)
           + hp_ref[...].astype(jnp.float32))
    xl = _leaky(agg * dinv + b_ref[...])
    g = jnp.mean(xl, axis=0, keepdims=True)
    z = _leaky(
        jnp.dot(g, fc1w_ref[...], preferred_element_type=jnp.float32)
        + fc1b_ref[...]
    )
    z = (
        jnp.dot(z, fc2w_ref[...], preferred_element_type=jnp.float32)
        + fc2b_ref[...]
    )
    out_ref[...] = 1.0 / (1.0 + jnp.exp(-z))


_tc_final = pl.pallas_call(
    _tc_final_body,
    out_shape=jax.ShapeDtypeStruct((1, 1), jnp.float32),
)


def kernel(x, edge_index, W1, b1, W2, b2, W3, b3, W4, b4,
           fc1_W, fc1_b, fc2_W, fc2_b):
    src2d = edge_index[0].reshape(NCHUNKS, CHUNK)
    dst2d = edge_index[1].reshape(NCHUNKS, CHUNK)
    zeros_deg = jnp.zeros((ROWS_PER_TILE, DEG_W), jnp.float32)
    ones_deg = jnp.ones((CHUNK, DEG_W), jnp.float32)
    zeros_h = jnp.zeros((CHUNK, D_H), jnp.bfloat16)

    degp = _sc_deg(dst2d, zeros_deg, ones_deg)
    hp, dinv = _tc_first(degp, x, W1)
    for b, w in ((b1, W2), (b2, W3), (b3, W4)):
        parts = _sc_scat(hp, src2d, dst2d, zeros_h)
        hp = _tc_mid(parts, hp, dinv, b.reshape(1, D_H), w)
    parts = _sc_scat(hp, src2d, dst2d, zeros_h)
    return _tc_final(
        parts, hp, dinv, b4.reshape(1, D_H),
        fc1_W, fc1_b.reshape(1, D_H), fc2_W, fc2_b.reshape(1, 1),
    )
